# baseline (speedup 1.0000x reference)
import functools

import jax
import jax.numpy as jnp
from jax import lax
from jax.experimental import pallas as pl
from jax.experimental.pallas import tpu as pltpu
from jax.experimental.pallas import tpu_sc as plsc

B = 4096
L = 200
E = 32
H = 64
TOT = B * L
NC, NS = 2, 16
NW = NC * NS
ROWS_PER_W = TOT // NW
CH = 128
NG = 10
GROUP = CH * NG
NGROUPS = ROWS_PER_W // GROUP
IDX_ROWS_PER_W = ROWS_PER_W // CH


def _sc_gather(table, idx2d, tot, ck):
    rows_per_w = tot // NW
    idx_rows_per_w = rows_per_w // CH
    ngroups = rows_per_w // GROUP
    assert ngroups % 2 == 0 and ngroups * GROUP == rows_per_w
    idx_base = ck * (tot // CH)
    mesh = plsc.VectorSubcoreMesh(core_axis_name="c", subcore_axis_name="s")

    @functools.partial(
        pl.kernel,
        mesh=mesh,
        out_type=jax.ShapeDtypeStruct((tot, E), jnp.bfloat16),
        scratch_types=[
            pltpu.VMEM((NG, CH), jnp.int32),
            pltpu.VMEM((NG, CH), jnp.int32),
            pltpu.VMEM((GROUP, E), jnp.bfloat16),
            pltpu.VMEM((GROUP, E), jnp.bfloat16),
            pltpu.SemaphoreType.DMA,
            pltpu.SemaphoreType.DMA,
            pltpu.SemaphoreType.DMA,
        ],
        compiler_params=pltpu.CompilerParams(use_tc_tiling_on_sc=False),
    )
    def k(table_hbm, idx_hbm, out_hbm, idx0, idx1, rows0, rows1, gsem,
          osem0, osem1):
        wid = lax.axis_index("s") * NC + lax.axis_index("c")
        row0 = wid * rows_per_w
        irow0 = idx_base + wid * idx_rows_per_w

        def half(g, idx_v, rows_v, osem, first):
            @pl.when(jnp.logical_not(first))
            def _():
                pltpu.make_async_copy(
                    rows_v, out_hbm.at[pl.ds(row0, GROUP)], osem).wait()
            pltpu.sync_copy(idx_hbm.at[pl.ds(irow0 + g * NG, NG)], idx_v)
            handles = []
            for j in range(NG):
                handles.append(
                    pltpu.async_copy(
                        table_hbm.at[idx_v.at[j]],
                        rows_v.at[pl.ds(j * CH, CH)],
                        gsem,
                    )
                )
            for h in handles:
                h.wait()
            pltpu.async_copy(
                rows_v, out_hbm.at[pl.ds(row0 + g * GROUP, GROUP)], osem)

        def body(i, carry):
            half(2 * i, idx0, rows0, osem0, i == 0)
            half(2 * i + 1, idx1, rows1, osem1, i == 0)
            return carry

        lax.fori_loop(0, ngroups // 2, body, 0)
        pltpu.make_async_copy(
            rows0, out_hbm.at[pl.ds(row0, GROUP)], osem0).wait()
        pltpu.make_async_copy(
            rows1, out_hbm.at[pl.ds(row0, GROUP)], osem1).wait()

    return k(table, idx2d)


PK = 128 // E
LP = L // PK
VOCAB = 1000000


VC = 32768
VQ = VC // PK
NBLK = pl.cdiv(VOCAB, VC)
TVOCAB = NBLK * VC


def _tc_pack_table(embT):
    def body(x0_ref, x1_ref, x2_ref, x3_ref, o_ref):
        xbig = jnp.concatenate(
            [x0_ref[...], x1_ref[...], x2_ref[...], x3_ref[...]], axis=0)
        o_ref[...] = jnp.transpose(xbig).astype(jnp.bfloat16)

    last_blk = (VOCAB - 1) // VQ

    def spec(k):
        return pl.BlockSpec(
            (E, VQ), lambda i, _k=k: (0, jnp.minimum(i * PK + _k, last_blk)))

    return pl.pallas_call(
        body,
        grid=(NBLK,),
        in_specs=[spec(0), spec(1), spec(2), spec(3)],
        out_specs=pl.BlockSpec((VQ, PK * E), lambda i: (i, 0)),
        out_shape=jax.ShapeDtypeStruct((TVOCAB // PK, PK * E), jnp.bfloat16),
    )(embT, embT, embT, embT)


def _permute_idx(v):
    blk = v // VC
    j = v % VC
    return blk * VC + (j % VQ) * PK + j // VQ


def _tc_mlp(g128, W1b, b1t, W2b, b2, W3, b3, W4, b4, bck):
    BB = 128
    NR = BB * LP

    def dotd(a, b):
        return jnp.dot(a.astype(jnp.bfloat16), b.astype(jnp.bfloat16),
                       preferred_element_type=jnp.float32)

    def body(x_ref, w1_ref, b1_ref, w2_ref, b2_ref, w3_ref, b3_ref, w4_ref,
             b4_ref, o_ref):
        x = x_ref[...]
        h = jnp.maximum(dotd(x, w1_ref[...]) + b1_ref[...], 0.0)
        h2 = dotd(h, w2_ref[...])
        pooled = jnp.mean(h2.reshape(BB, LP, PK * H), axis=1)
        p = 0.25 * (pooled[:, 0:H] + pooled[:, H:2 * H] +
                    pooled[:, 2 * H:3 * H] + pooled[:, 3 * H:4 * H])
        p = p + b2_ref[...]
        q = jnp.maximum(dotd(p, w3_ref[...]) + b3_ref[...], 0.0)
        o_ref[...] = dotd(q, w4_ref[...]) + b4_ref[...]

    full = lambda shape: pl.BlockSpec(shape, lambda i: (0,) * len(shape))
    return pl.pallas_call(
        body,
        grid=(bck // BB,),
        in_specs=[
            pl.BlockSpec((NR, 128), lambda i: (i, 0)),
            full((128, PK * H)), full((1, PK * H)),
            full((PK * H, PK * H)), full((1, H)),
            full((H, H)), full((1, H)),
            full((H, 1)), full((1, 1)),
        ],
        out_specs=pl.BlockSpec((BB, 1), lambda i: (i, 0)),
        out_shape=jax.ShapeDtypeStruct((bck, 1), jnp.float32),
    )(g128, W1b, b1t, W2b, b2.reshape(1, H),
      W3, b3.reshape(1, H), W4, b4.reshape(1, 1))


NCK = 2


def kernel(events, emb, W1, b1, W2, b2, W3, b3, W4, b4):
    idx2d = _permute_idx(events.astype(jnp.int32)).reshape(TOT // CH, CH)
    table = _tc_pack_table(emb.T).reshape(TVOCAB, E)
    W1b = jnp.zeros((128, PK * H), dtype=jnp.float32)
    W2b = jnp.zeros((PK * H, PK * H), dtype=jnp.float32)
    for k in range(PK):
        W1b = W1b.at[k * E:(k + 1) * E, k * H:(k + 1) * H].set(W1)
        W2b = W2b.at[k * H:(k + 1) * H, k * H:(k + 1) * H].set(W2)
    b1t = jnp.tile(b1, PK).reshape(1, PK * H)
    tot_ck = TOT // NCK
    outs = []
    for ck in range(NCK):
        g = _sc_gather(table, idx2d, tot_ck, ck)
        g128 = g.reshape(tot_ck // PK, 128)
        outs.append(
            _tc_mlp(g128, W1b, b1t, W2b, b2, W3, b3, W4, b4, B // NCK))
    return jnp.concatenate(outs, axis=0)


# speedup vs baseline: 2.4129x; 2.4129x over previous
import functools

import jax
import jax.numpy as jnp
from jax import lax
from jax.experimental import pallas as pl
from jax.experimental.pallas import tpu as pltpu
from jax.experimental.pallas import tpu_sc as plsc

B = 4096
L = 200
E = 32
H = 64
TOT = B * L
NC, NS = 2, 16
NW = NC * NS
ROWS_PER_W = TOT // NW
CH = 128
NG = 10
GROUP = CH * NG
NGROUPS = ROWS_PER_W // GROUP
IDX_ROWS_PER_W = ROWS_PER_W // CH


def _sc_gather(table, idx2d, tot, ck):
    rows_per_w = tot // NW
    idx_rows_per_w = rows_per_w // CH
    ngroups = rows_per_w // GROUP
    assert ngroups % 2 == 0 and ngroups * GROUP == rows_per_w
    idx_base = ck * (tot // CH)
    mesh = plsc.VectorSubcoreMesh(core_axis_name="c", subcore_axis_name="s")

    @functools.partial(
        pl.kernel,
        mesh=mesh,
        out_type=jax.ShapeDtypeStruct((tot, E), jnp.float32),
        scratch_types=[
            pltpu.VMEM((NG, CH), jnp.int32),
            pltpu.VMEM((NG, CH), jnp.int32),
            pltpu.VMEM((GROUP, E), jnp.float32),
            pltpu.VMEM((GROUP, E), jnp.float32),
            pltpu.SemaphoreType.DMA,
            pltpu.SemaphoreType.DMA,
            pltpu.SemaphoreType.DMA,
        ],
        compiler_params=pltpu.CompilerParams(use_tc_tiling_on_sc=False),
    )
    def k(table_hbm, idx_hbm, out_hbm, idx0, idx1, rows0, rows1, gsem,
          osem0, osem1):
        wid = lax.axis_index("s") * NC + lax.axis_index("c")
        row0 = wid * rows_per_w
        irow0 = idx_base + wid * idx_rows_per_w

        def half(g, idx_v, rows_v, osem, first):
            @pl.when(jnp.logical_not(first))
            def _():
                pltpu.make_async_copy(
                    rows_v, out_hbm.at[pl.ds(row0, GROUP)], osem).wait()
            pltpu.sync_copy(idx_hbm.at[pl.ds(irow0 + g * NG, NG)], idx_v)
            handles = []
            for j in range(NG):
                handles.append(
                    pltpu.async_copy(
                        table_hbm.at[idx_v.at[j]],
                        rows_v.at[pl.ds(j * CH, CH)],
                        gsem,
                    )
                )
            for h in handles:
                h.wait()
            pltpu.async_copy(
                rows_v, out_hbm.at[pl.ds(row0 + g * GROUP, GROUP)], osem)

        def body(i, carry):
            half(2 * i, idx0, rows0, osem0, i == 0)
            half(2 * i + 1, idx1, rows1, osem1, i == 0)
            return carry

        lax.fori_loop(0, ngroups // 2, body, 0)
        pltpu.make_async_copy(
            rows0, out_hbm.at[pl.ds(row0, GROUP)], osem0).wait()
        pltpu.make_async_copy(
            rows1, out_hbm.at[pl.ds(row0, GROUP)], osem1).wait()

    return k(table, idx2d)


PK = 128 // E
LP = L // PK
VOCAB = 1000000


VC = 32768
VQ = VC // PK
NBLK = pl.cdiv(VOCAB, VC)
TVOCAB = NBLK * VC


def _tc_pack_table(embT):
    def body(x0_ref, x1_ref, x2_ref, x3_ref, o_ref):
        xbig = jnp.concatenate(
            [x0_ref[...], x1_ref[...], x2_ref[...], x3_ref[...]], axis=0)
        o_ref[...] = jnp.transpose(xbig)

    last_blk = (VOCAB - 1) // VQ

    def spec(k):
        return pl.BlockSpec(
            (E, VQ), lambda i, _k=k: (0, jnp.minimum(i * PK + _k, last_blk)))

    return pl.pallas_call(
        body,
        grid=(NBLK,),
        in_specs=[spec(0), spec(1), spec(2), spec(3)],
        out_specs=pl.BlockSpec((VQ, PK * E), lambda i: (i, 0)),
        out_shape=jax.ShapeDtypeStruct((TVOCAB // PK, PK * E), jnp.float32),
    )(embT, embT, embT, embT)


def _permute_idx(v):
    blk = v // VC
    j = v % VC
    return blk * VC + (j % VQ) * PK + j // VQ


def _tc_mlp(g128, W1b, b1t, W2b, b2, W3, b3, W4, b4, bck):
    BB = 128
    NR = BB * LP

    def dotd(a, b):
        return jnp.dot(a.astype(jnp.bfloat16), b.astype(jnp.bfloat16),
                       preferred_element_type=jnp.float32)

    def body(x_ref, w1_ref, b1_ref, w2_ref, b2_ref, w3_ref, b3_ref, w4_ref,
             b4_ref, o_ref):
        x = x_ref[...]
        h = jnp.maximum(dotd(x, w1_ref[...]) + b1_ref[...], 0.0)
        h2 = dotd(h, w2_ref[...])
        pooled = jnp.mean(h2.reshape(BB, LP, PK * H), axis=1)
        p = 0.25 * (pooled[:, 0:H] + pooled[:, H:2 * H] +
                    pooled[:, 2 * H:3 * H] + pooled[:, 3 * H:4 * H])
        p = p + b2_ref[...]
        q = jnp.maximum(dotd(p, w3_ref[...]) + b3_ref[...], 0.0)
        o_ref[...] = dotd(q, w4_ref[...]) + b4_ref[...]

    full = lambda shape: pl.BlockSpec(shape, lambda i: (0,) * len(shape))
    return pl.pallas_call(
        body,
        grid=(bck // BB,),
        in_specs=[
            pl.BlockSpec((NR, 128), lambda i: (i, 0)),
            full((128, PK * H)), full((1, PK * H)),
            full((PK * H, PK * H)), full((1, H)),
            full((H, H)), full((1, H)),
            full((H, 1)), full((1, 1)),
        ],
        out_specs=pl.BlockSpec((BB, 1), lambda i: (i, 0)),
        out_shape=jax.ShapeDtypeStruct((bck, 1), jnp.float32),
    )(g128, W1b, b1t, W2b, b2.reshape(1, H),
      W3, b3.reshape(1, H), W4, b4.reshape(1, 1))


NCK = 2


def kernel(events, emb, W1, b1, W2, b2, W3, b3, W4, b4):
    idx2d = _permute_idx(events.astype(jnp.int32)).reshape(TOT // CH, CH)
    table = _tc_pack_table(emb.T).reshape(TVOCAB, E)
    W1b = jnp.zeros((128, PK * H), dtype=jnp.float32)
    W2b = jnp.zeros((PK * H, PK * H), dtype=jnp.float32)
    for k in range(PK):
        W1b = W1b.at[k * E:(k + 1) * E, k * H:(k + 1) * H].set(W1)
        W2b = W2b.at[k * H:(k + 1) * H, k * H:(k + 1) * H].set(W2)
    b1t = jnp.tile(b1, PK).reshape(1, PK * H)
    tot_ck = TOT // NCK
    outs = []
    for ck in range(NCK):
        g = _sc_gather(table, idx2d, tot_ck, ck)
        g128 = g.reshape(tot_ck // PK, 128)
        outs.append(
            _tc_mlp(g128, W1b, b1t, W2b, b2, W3, b3, W4, b4, B // NCK))
    return jnp.concatenate(outs, axis=0)
